# Initial kernel scaffold; baseline (speedup 1.0000x reference)
import functools

import jax
import jax.numpy as jnp
from jax import lax
from jax.experimental import pallas as pl
from jax.experimental.pallas import tpu as pltpu
from jax.experimental.pallas import tpu_sc as plsc

N = 10000
E = 320000
D = 128

NC = 2
NS = 16
NW = NC * NS
EPW = E // NW
CHUNK = 80
NCHUNK = EPW // CHUNK
RPS = N // NS
RSTEP = 125

_mesh = plsc.VectorSubcoreMesh(core_axis_name="c", subcore_axis_name="s")


def _proj_body(x_ref, w1_ref, b1_ref, xs_ref, xr_ref):
    x = x_ref[...]
    xs_ref[...] = (
        jnp.dot(x, w1_ref[0:D, :], preferred_element_type=jnp.float32)
        + b1_ref[...]
    )
    xr_ref[...] = jnp.dot(x, w1_ref[D:2 * D, :], preferred_element_type=jnp.float32)


def _edge_body(gs_ref, gr_ref, e_ref, w1e_ref, w2_ref, b2_ref, ne_ref, eo_ref):
    e = e_ref[...]
    pe = jnp.dot(e, w1e_ref[...], preferred_element_type=jnp.float32)
    h = jnp.maximum(gs_ref[...] + gr_ref[...] + pe, 0.0)
    tmp = jnp.dot(h, w2_ref[...], preferred_element_type=jnp.float32) + b2_ref[...]
    ne_ref[...] = tmp
    eo_ref[...] = tmp + e


def _node_body(x_ref, p_ref, w1_ref, b1_ref, w2_ref, b2_ref, out_ref):
    x = x_ref[...]
    agg = p_ref[0] + p_ref[1]
    h = jnp.maximum(
        jnp.dot(x, w1_ref[0:D, :], preferred_element_type=jnp.float32)
        + jnp.dot(agg, w1_ref[D:2 * D, :], preferred_element_type=jnp.float32)
        + b1_ref[...],
        0.0,
    )
    out_ref[...] = (
        jnp.dot(h, w2_ref[...], preferred_element_type=jnp.float32)
        + b2_ref[...]
        + x
    )


@functools.partial(
    pl.kernel,
    mesh=_mesh,
    out_type=[
        jax.ShapeDtypeStruct((E, D), jnp.float32),
        jax.ShapeDtypeStruct((E, D), jnp.float32),
    ],
    scratch_types=[
        pltpu.VMEM((CHUNK,), jnp.int32),
        pltpu.VMEM((CHUNK,), jnp.int32),
        pltpu.VMEM((CHUNK, D), jnp.float32),
        pltpu.VMEM((CHUNK, D), jnp.float32),
        pltpu.SemaphoreType.DMA,
        pltpu.SemaphoreType.DMA,
    ],
)
def _gather_sc(xs_hbm, xr_hbm, snd_hbm, rcv_hbm, gs_hbm, gr_hbm,
               idx_s, idx_r, rows_s, rows_r, sem_s, sem_r):
    wid = lax.axis_index("s") * NC + lax.axis_index("c")
    base = wid * EPW

    def body(k, carry):
        off = base + k * CHUNK
        pltpu.sync_copy(snd_hbm.at[pl.ds(off, CHUNK)], idx_s)
        pltpu.sync_copy(rcv_hbm.at[pl.ds(off, CHUNK)], idx_r)
        cs = pltpu.async_copy(xs_hbm.at[idx_s], rows_s, sem_s)
        cr = pltpu.async_copy(xr_hbm.at[idx_r], rows_r, sem_r)
        cs.wait()
        cr.wait()
        pltpu.sync_copy(rows_s, gs_hbm.at[pl.ds(off, CHUNK)])
        pltpu.sync_copy(rows_r, gr_hbm.at[pl.ds(off, CHUNK)])
        return carry

    lax.fori_loop(0, NCHUNK, body, 0)


@functools.partial(
    pl.kernel,
    mesh=_mesh,
    out_type=jax.ShapeDtypeStruct((NC, N, D), jnp.float32),
    scratch_types=[
        pltpu.VMEM((CHUNK,), jnp.int32),
        pltpu.VMEM((CHUNK, D), jnp.float32),
        pltpu.VMEM((RSTEP, D), jnp.float32),
        pltpu.VMEM_SHARED((N, D), jnp.float32),
    ],
)
def _scatter_sc(ne_hbm, rcv_hbm, zero_hbm, parts_hbm, idx, chunk, stage, acc):
    cid = lax.axis_index("c")
    sid = lax.axis_index("s")
    wid = sid * NC + cid

    def zbody(i, carry):
        r0 = sid * RPS + i * RSTEP
        pltpu.sync_copy(zero_hbm.at[pl.ds(r0, RSTEP)], stage)
        pltpu.sync_copy(stage, acc.at[pl.ds(r0, RSTEP)])
        return carry

    lax.fori_loop(0, RPS // RSTEP, zbody, 0)
    plsc.subcore_barrier()

    base = wid * EPW

    def body(k, carry):
        off = base + k * CHUNK
        pltpu.sync_copy(rcv_hbm.at[pl.ds(off, CHUNK)], idx)
        pltpu.sync_copy(ne_hbm.at[pl.ds(off, CHUNK)], chunk)
        pltpu.sync_copy(chunk, acc.at[idx], add=True)
        return carry

    lax.fori_loop(0, NCHUNK, body, 0)
    plsc.subcore_barrier()

    def obody(i, carry):
        r0 = sid * RPS + i * RSTEP
        pltpu.sync_copy(acc.at[pl.ds(r0, RSTEP)], stage)
        pltpu.sync_copy(stage, parts_hbm.at[cid, pl.ds(r0, RSTEP)])
        return carry

    lax.fori_loop(0, RPS // RSTEP, obody, 0)


def kernel(node_features, edge_features, me_w1, me_b1, me_w2, me_b2,
           nm_w1, nm_b1, nm_w2, nm_b2, senders, receivers):
    snd = senders.astype(jnp.int32)
    rcv = receivers.astype(jnp.int32)

    BN = 1000
    xs, xr = pl.pallas_call(
        _proj_body,
        grid=(N // BN,),
        in_specs=[
            pl.BlockSpec((BN, D), lambda i: (i, 0)),
            pl.BlockSpec((3 * D, D), lambda i: (0, 0)),
            pl.BlockSpec((1, D), lambda i: (0, 0)),
        ],
        out_specs=[pl.BlockSpec((BN, D), lambda i: (i, 0))] * 2,
        out_shape=[jax.ShapeDtypeStruct((N, D), jnp.float32)] * 2,
    )(node_features, me_w1, me_b1.reshape(1, D))

    gs, gr = _gather_sc(xs, xr, snd, rcv)

    BE = 2000
    ne, edge_out = pl.pallas_call(
        _edge_body,
        grid=(E // BE,),
        in_specs=[
            pl.BlockSpec((BE, D), lambda i: (i, 0)),
            pl.BlockSpec((BE, D), lambda i: (i, 0)),
            pl.BlockSpec((BE, D), lambda i: (i, 0)),
            pl.BlockSpec((D, D), lambda i: (0, 0)),
            pl.BlockSpec((D, D), lambda i: (0, 0)),
            pl.BlockSpec((1, D), lambda i: (0, 0)),
        ],
        out_specs=[pl.BlockSpec((BE, D), lambda i: (i, 0))] * 2,
        out_shape=[jax.ShapeDtypeStruct((E, D), jnp.float32)] * 2,
    )(gs, gr, edge_features, me_w1[2 * D:3 * D], me_w2, me_b2.reshape(1, D))

    zeros = jnp.zeros((N, D), jnp.float32)
    parts = _scatter_sc(ne, rcv, zeros)

    node_out = pl.pallas_call(
        _node_body,
        grid=(N // BN,),
        in_specs=[
            pl.BlockSpec((BN, D), lambda i: (i, 0)),
            pl.BlockSpec((NC, BN, D), lambda i: (0, i, 0)),
            pl.BlockSpec((2 * D, D), lambda i: (0, 0)),
            pl.BlockSpec((1, D), lambda i: (0, 0)),
            pl.BlockSpec((D, D), lambda i: (0, 0)),
            pl.BlockSpec((1, D), lambda i: (0, 0)),
        ],
        out_specs=pl.BlockSpec((BN, D), lambda i: (i, 0)),
        out_shape=jax.ShapeDtypeStruct((N, D), jnp.float32),
    )(node_features, parts, nm_w1, nm_b1.reshape(1, D), nm_w2, nm_b2.reshape(1, D))

    return node_out, edge_out


# speedup vs baseline: 2.8141x; 2.8141x over previous
import functools

import jax
import jax.numpy as jnp
from jax import lax
from jax.experimental import pallas as pl
from jax.experimental.pallas import tpu as pltpu
from jax.experimental.pallas import tpu_sc as plsc

N = 10000
E = 320000
D = 128

NC = 2
NS = 16
NW = NC * NS
EPW = E // NW
CHUNK = 80
NCHUNK = EPW // CHUNK
NP = 10240
RPS = NP // NS
RSTEP = 128

_mesh = plsc.VectorSubcoreMesh(core_axis_name="c", subcore_axis_name="s")


def _proj_body(x_ref, w1_ref, b1_ref, xs_ref, xr_ref):
    x = x_ref[...]
    xs_ref[...] = (
        jnp.dot(x, w1_ref[0:D, :], preferred_element_type=jnp.float32)
        + b1_ref[...]
    )
    xr_ref[...] = jnp.dot(x, w1_ref[D:2 * D, :], preferred_element_type=jnp.float32)


def _edge_body(gs_ref, gr_ref, e_ref, w1e_ref, w2_ref, b2_ref, ne_ref, eo_ref):
    e = e_ref[...]
    pe = jnp.dot(e, w1e_ref[...], preferred_element_type=jnp.float32)
    h = jnp.maximum(gs_ref[...] + gr_ref[...] + pe, 0.0)
    tmp = jnp.dot(h, w2_ref[...], preferred_element_type=jnp.float32) + b2_ref[...]
    ne_ref[...] = tmp
    eo_ref[...] = tmp + e


def _node_body(x_ref, p_ref, w1_ref, b1_ref, w2_ref, b2_ref, out_ref):
    x = x_ref[...]
    agg = p_ref[0] + p_ref[1]
    h = jnp.maximum(
        jnp.dot(x, w1_ref[0:D, :], preferred_element_type=jnp.float32)
        + jnp.dot(agg, w1_ref[D:2 * D, :], preferred_element_type=jnp.float32)
        + b1_ref[...],
        0.0,
    )
    out_ref[...] = (
        jnp.dot(h, w2_ref[...], preferred_element_type=jnp.float32)
        + b2_ref[...]
        + x
    )


@functools.partial(
    pl.kernel,
    mesh=_mesh,
    out_type=[
        jax.ShapeDtypeStruct((E, D), jnp.float32),
        jax.ShapeDtypeStruct((E, D), jnp.float32),
    ],
    scratch_types=[
        pltpu.VMEM((CHUNK,), jnp.int32),
        pltpu.VMEM((CHUNK,), jnp.int32),
        pltpu.VMEM((CHUNK, D), jnp.float32),
        pltpu.VMEM((CHUNK, D), jnp.float32),
        pltpu.SemaphoreType.DMA,
        pltpu.SemaphoreType.DMA,
    ],
)
def _gather_sc(xs_hbm, xr_hbm, snd_hbm, rcv_hbm, gs_hbm, gr_hbm,
               idx_s, idx_r, rows_s, rows_r, sem_s, sem_r):
    wid = lax.axis_index("s") * NC + lax.axis_index("c")
    base = wid * EPW

    def body(k, carry):
        off = base + k * CHUNK
        pltpu.sync_copy(snd_hbm.at[pl.ds(off, CHUNK)], idx_s)
        pltpu.sync_copy(rcv_hbm.at[pl.ds(off, CHUNK)], idx_r)
        cs = pltpu.async_copy(xs_hbm.at[idx_s], rows_s, sem_s)
        cr = pltpu.async_copy(xr_hbm.at[idx_r], rows_r, sem_r)
        cs.wait()
        cr.wait()
        pltpu.sync_copy(rows_s, gs_hbm.at[pl.ds(off, CHUNK)])
        pltpu.sync_copy(rows_r, gr_hbm.at[pl.ds(off, CHUNK)])
        return carry

    lax.fori_loop(0, NCHUNK, body, 0)


@functools.partial(
    pl.kernel,
    mesh=_mesh,
    out_type=jax.ShapeDtypeStruct((NC, NP, D), jnp.float32),
    scratch_types=[
        pltpu.VMEM((CHUNK,), jnp.int32),
        pltpu.VMEM((CHUNK, D), jnp.float32),
        pltpu.VMEM((RSTEP, D), jnp.float32),
        pltpu.VMEM_SHARED((NP, D), jnp.float32),
    ],
)
def _scatter_sc(ne_hbm, rcv_hbm, zero_hbm, parts_hbm, idx, chunk, stage, acc):
    cid = lax.axis_index("c")
    sid = lax.axis_index("s")
    wid = sid * NC + cid

    def zbody(i, carry):
        r0 = sid * RPS + i * RSTEP
        pltpu.sync_copy(zero_hbm.at[pl.ds(r0, RSTEP)], stage)
        pltpu.sync_copy(stage, acc.at[pl.ds(r0, RSTEP)])
        return carry

    lax.fori_loop(0, RPS // RSTEP, zbody, 0)
    plsc.subcore_barrier()

    base = wid * EPW

    def body(k, carry):
        off = base + k * CHUNK
        pltpu.sync_copy(rcv_hbm.at[pl.ds(off, CHUNK)], idx)
        pltpu.sync_copy(ne_hbm.at[pl.ds(off, CHUNK)], chunk)
        pltpu.sync_copy(chunk, acc.at[idx], add=True)
        return carry

    lax.fori_loop(0, NCHUNK, body, 0)
    plsc.subcore_barrier()

    def obody(i, carry):
        r0 = sid * RPS + i * RSTEP
        pltpu.sync_copy(acc.at[pl.ds(r0, RSTEP)], stage)
        pltpu.sync_copy(stage, parts_hbm.at[cid, pl.ds(r0, RSTEP)])
        return carry

    lax.fori_loop(0, RPS // RSTEP, obody, 0)


def kernel(node_features, edge_features, me_w1, me_b1, me_w2, me_b2,
           nm_w1, nm_b1, nm_w2, nm_b2, senders, receivers):
    snd = senders.astype(jnp.int32)
    rcv = receivers.astype(jnp.int32)

    BN = 1000
    xs, xr = pl.pallas_call(
        _proj_body,
        grid=(N // BN,),
        in_specs=[
            pl.BlockSpec((BN, D), lambda i: (i, 0)),
            pl.BlockSpec((3 * D, D), lambda i: (0, 0)),
            pl.BlockSpec((1, D), lambda i: (0, 0)),
        ],
        out_specs=[pl.BlockSpec((BN, D), lambda i: (i, 0))] * 2,
        out_shape=[jax.ShapeDtypeStruct((N, D), jnp.float32)] * 2,
    )(node_features, me_w1, me_b1.reshape(1, D))

    gs, gr = _gather_sc(xs, xr, snd, rcv)

    BE = 2000
    ne, edge_out = pl.pallas_call(
        _edge_body,
        grid=(E // BE,),
        in_specs=[
            pl.BlockSpec((BE, D), lambda i: (i, 0)),
            pl.BlockSpec((BE, D), lambda i: (i, 0)),
            pl.BlockSpec((BE, D), lambda i: (i, 0)),
            pl.BlockSpec((D, D), lambda i: (0, 0)),
            pl.BlockSpec((D, D), lambda i: (0, 0)),
            pl.BlockSpec((1, D), lambda i: (0, 0)),
        ],
        out_specs=[pl.BlockSpec((BE, D), lambda i: (i, 0))] * 2,
        out_shape=[jax.ShapeDtypeStruct((E, D), jnp.float32)] * 2,
    )(gs, gr, edge_features, me_w1[2 * D:3 * D], me_w2, me_b2.reshape(1, D))

    zeros = jnp.zeros((NP, D), jnp.float32)
    parts = _scatter_sc(ne, rcv, zeros)

    node_out = pl.pallas_call(
        _node_body,
        grid=(N // BN,),
        in_specs=[
            pl.BlockSpec((BN, D), lambda i: (i, 0)),
            pl.BlockSpec((NC, BN, D), lambda i: (0, i, 0)),
            pl.BlockSpec((2 * D, D), lambda i: (0, 0)),
            pl.BlockSpec((1, D), lambda i: (0, 0)),
            pl.BlockSpec((D, D), lambda i: (0, 0)),
            pl.BlockSpec((1, D), lambda i: (0, 0)),
        ],
        out_specs=pl.BlockSpec((BN, D), lambda i: (i, 0)),
        out_shape=jax.ShapeDtypeStruct((N, D), jnp.float32),
    )(node_features, parts, nm_w1, nm_b1.reshape(1, D), nm_w2, nm_b2.reshape(1, D))

    return node_out, edge_out


# speedup vs baseline: 4.1324x; 1.4685x over previous
import functools

import jax
import jax.numpy as jnp
from jax import lax
from jax.experimental import pallas as pl
from jax.experimental.pallas import tpu as pltpu
from jax.experimental.pallas import tpu_sc as plsc

N = 10000
E = 320000
D = 128

NC = 2
NS = 16
NW = NC * NS
EPW = E // NW
CHUNK = 80
NCHUNK = EPW // CHUNK
NBUF = 5
NOUTER = NCHUNK // NBUF
NP = 10240
RPS = NP // NS
SCHUNK = 40
SNCHUNK = EPW // SCHUNK
SNOUTER = SNCHUNK // NBUF
RSTEP = SCHUNK

_mesh = plsc.VectorSubcoreMesh(core_axis_name="c", subcore_axis_name="s")


def _proj_body(x_ref, w1_ref, b1_ref, xs_ref, xr_ref):
    x = x_ref[...]
    xs_ref[...] = (
        jnp.dot(x, w1_ref[0:D, :], preferred_element_type=jnp.float32)
        + b1_ref[...]
    )
    xr_ref[...] = jnp.dot(x, w1_ref[D:2 * D, :], preferred_element_type=jnp.float32)


def _edge_body(gs_ref, gr_ref, e_ref, w1e_ref, w2_ref, b2_ref, ne_ref, eo_ref):
    e = e_ref[...]
    pe = jnp.dot(e, w1e_ref[...], preferred_element_type=jnp.float32)
    h = jnp.maximum(gs_ref[...] + gr_ref[...] + pe, 0.0)
    tmp = jnp.dot(h, w2_ref[...], preferred_element_type=jnp.float32) + b2_ref[...]
    ne_ref[...] = tmp
    eo_ref[...] = tmp + e


def _node_body(x_ref, p_ref, w1_ref, b1_ref, w2_ref, b2_ref, out_ref):
    x = x_ref[...]
    agg = p_ref[0] + p_ref[1]
    h = jnp.maximum(
        jnp.dot(x, w1_ref[0:D, :], preferred_element_type=jnp.float32)
        + jnp.dot(agg, w1_ref[D:2 * D, :], preferred_element_type=jnp.float32)
        + b1_ref[...],
        0.0,
    )
    out_ref[...] = (
        jnp.dot(h, w2_ref[...], preferred_element_type=jnp.float32)
        + b2_ref[...]
        + x
    )


@functools.partial(
    pl.kernel,
    mesh=_mesh,
    out_type=[
        jax.ShapeDtypeStruct((E, D), jnp.float32),
        jax.ShapeDtypeStruct((E, D), jnp.float32),
    ],
    scratch_types=(
        [pltpu.VMEM((CHUNK,), jnp.int32) for _ in range(2 * NBUF)]
        + [pltpu.VMEM((CHUNK, D), jnp.float32) for _ in range(2 * NBUF)]
        + [pltpu.SemaphoreType.DMA for _ in range(3 * NBUF)]
    ),
)
def _gather_sc(xs_hbm, xr_hbm, snd_hbm, rcv_hbm, gs_hbm, gr_hbm, *scr):
    idx_s = scr[0:NBUF]
    idx_r = scr[NBUF:2 * NBUF]
    rows_s = scr[2 * NBUF:3 * NBUF]
    rows_r = scr[3 * NBUF:4 * NBUF]
    sem_i = scr[4 * NBUF:5 * NBUF]
    sem_g = scr[5 * NBUF:6 * NBUF]
    sem_w = scr[6 * NBUF:7 * NBUF]

    wid = lax.axis_index("s") * NC + lax.axis_index("c")
    base = wid * EPW

    for b in range(NBUF):
        off = base + b * CHUNK
        pltpu.async_copy(snd_hbm.at[pl.ds(off, CHUNK)], idx_s[b], sem_i[b])
        pltpu.async_copy(rcv_hbm.at[pl.ds(off, CHUNK)], idx_r[b], sem_i[b])

    def outer(g, carry):
        for b in range(NBUF):
            k = g * NBUF + b
            off = base + k * CHUNK

            @pl.when(g > 0)
            def _drain():
                pltpu.make_async_copy(
                    rows_s[b], gs_hbm.at[pl.ds(base, CHUNK)], sem_w[b]).wait()
                pltpu.make_async_copy(
                    rows_r[b], gr_hbm.at[pl.ds(base, CHUNK)], sem_w[b]).wait()

            pltpu.make_async_copy(
                snd_hbm.at[pl.ds(base, CHUNK)], idx_s[b], sem_i[b]).wait()
            pltpu.make_async_copy(
                rcv_hbm.at[pl.ds(base, CHUNK)], idx_r[b], sem_i[b]).wait()
            cs = pltpu.async_copy(xs_hbm.at[idx_s[b]], rows_s[b], sem_g[b])
            cr = pltpu.async_copy(xr_hbm.at[idx_r[b]], rows_r[b], sem_g[b])
            cs.wait()
            cr.wait()
            pltpu.async_copy(rows_s[b], gs_hbm.at[pl.ds(off, CHUNK)], sem_w[b])
            pltpu.async_copy(rows_r[b], gr_hbm.at[pl.ds(off, CHUNK)], sem_w[b])

            @pl.when(k + NBUF < NCHUNK)
            def _prefetch():
                off2 = off + NBUF * CHUNK
                pltpu.async_copy(snd_hbm.at[pl.ds(off2, CHUNK)], idx_s[b], sem_i[b])
                pltpu.async_copy(rcv_hbm.at[pl.ds(off2, CHUNK)], idx_r[b], sem_i[b])
        return carry

    lax.fori_loop(0, NOUTER, outer, 0)
    for b in range(NBUF):
        pltpu.make_async_copy(
            rows_s[b], gs_hbm.at[pl.ds(base, CHUNK)], sem_w[b]).wait()
        pltpu.make_async_copy(
            rows_r[b], gr_hbm.at[pl.ds(base, CHUNK)], sem_w[b]).wait()


@functools.partial(
    pl.kernel,
    mesh=_mesh,
    out_type=jax.ShapeDtypeStruct((NC, NP, D), jnp.float32),
    scratch_types=(
        [pltpu.VMEM((SCHUNK,), jnp.int32) for _ in range(NBUF)]
        + [pltpu.VMEM((SCHUNK, D), jnp.float32) for _ in range(NBUF)]
        + [pltpu.VMEM_SHARED((NP, D), jnp.float32)]
        + [pltpu.SemaphoreType.DMA for _ in range(NBUF)]
    ),
)
def _scatter_sc(ne_hbm, rcv_hbm, zero_hbm, parts_hbm, *scr):
    idx = scr[0:NBUF]
    chunk = scr[NBUF:2 * NBUF]
    acc = scr[2 * NBUF]
    sem_i = scr[2 * NBUF + 1:3 * NBUF + 1]

    cid = lax.axis_index("c")
    sid = lax.axis_index("s")
    wid = sid * NC + cid

    pltpu.sync_copy(zero_hbm.at[pl.ds(0, RSTEP)], chunk[0])

    def zbody(i, carry):
        r0 = sid * RPS + i * RSTEP
        pltpu.sync_copy(chunk[0], acc.at[pl.ds(r0, RSTEP)])
        return carry

    lax.fori_loop(0, RPS // RSTEP, zbody, 0)
    plsc.subcore_barrier()

    base = wid * EPW

    for b in range(NBUF):
        off = base + b * SCHUNK
        pltpu.async_copy(rcv_hbm.at[pl.ds(off, SCHUNK)], idx[b], sem_i[b])
        pltpu.async_copy(ne_hbm.at[pl.ds(off, SCHUNK)], chunk[b], sem_i[b])

    def outer(g, carry):
        for b in range(NBUF):
            k = g * NBUF + b
            off = base + k * SCHUNK
            pltpu.make_async_copy(
                rcv_hbm.at[pl.ds(base, SCHUNK)], idx[b], sem_i[b]).wait()
            pltpu.make_async_copy(
                ne_hbm.at[pl.ds(base, SCHUNK)], chunk[b], sem_i[b]).wait()
            pltpu.sync_copy(chunk[b], acc.at[idx[b]], add=True)

            @pl.when(k + NBUF < SNCHUNK)
            def _prefetch():
                off2 = off + NBUF * SCHUNK
                pltpu.async_copy(rcv_hbm.at[pl.ds(off2, SCHUNK)], idx[b], sem_i[b])
                pltpu.async_copy(ne_hbm.at[pl.ds(off2, SCHUNK)], chunk[b], sem_i[b])
        return carry

    lax.fori_loop(0, SNOUTER, outer, 0)
    plsc.subcore_barrier()

    def obody(i, carry):
        r0 = sid * RPS + i * RSTEP
        pltpu.sync_copy(acc.at[pl.ds(r0, RSTEP)], chunk[0])
        pltpu.sync_copy(chunk[0], parts_hbm.at[cid, pl.ds(r0, RSTEP)])
        return carry

    lax.fori_loop(0, RPS // RSTEP, obody, 0)


def kernel(node_features, edge_features, me_w1, me_b1, me_w2, me_b2,
           nm_w1, nm_b1, nm_w2, nm_b2, senders, receivers):
    snd = senders.astype(jnp.int32)
    rcv = receivers.astype(jnp.int32)

    BN = 1000
    xs, xr = pl.pallas_call(
        _proj_body,
        grid=(N // BN,),
        in_specs=[
            pl.BlockSpec((BN, D), lambda i: (i, 0)),
            pl.BlockSpec((3 * D, D), lambda i: (0, 0)),
            pl.BlockSpec((1, D), lambda i: (0, 0)),
        ],
        out_specs=[pl.BlockSpec((BN, D), lambda i: (i, 0))] * 2,
        out_shape=[jax.ShapeDtypeStruct((N, D), jnp.float32)] * 2,
    )(node_features, me_w1, me_b1.reshape(1, D))

    gs, gr = _gather_sc(xs, xr, snd, rcv)

    BE = 2000
    ne, edge_out = pl.pallas_call(
        _edge_body,
        grid=(E // BE,),
        in_specs=[
            pl.BlockSpec((BE, D), lambda i: (i, 0)),
            pl.BlockSpec((BE, D), lambda i: (i, 0)),
            pl.BlockSpec((BE, D), lambda i: (i, 0)),
            pl.BlockSpec((D, D), lambda i: (0, 0)),
            pl.BlockSpec((D, D), lambda i: (0, 0)),
            pl.BlockSpec((1, D), lambda i: (0, 0)),
        ],
        out_specs=[pl.BlockSpec((BE, D), lambda i: (i, 0))] * 2,
        out_shape=[jax.ShapeDtypeStruct((E, D), jnp.float32)] * 2,
    )(gs, gr, edge_features, me_w1[2 * D:3 * D], me_w2, me_b2.reshape(1, D))

    zeros = jnp.zeros((NP, D), jnp.float32)
    parts = _scatter_sc(ne, rcv, zeros)

    node_out = pl.pallas_call(
        _node_body,
        grid=(N // BN,),
        in_specs=[
            pl.BlockSpec((BN, D), lambda i: (i, 0)),
            pl.BlockSpec((NC, BN, D), lambda i: (0, i, 0)),
            pl.BlockSpec((2 * D, D), lambda i: (0, 0)),
            pl.BlockSpec((1, D), lambda i: (0, 0)),
            pl.BlockSpec((D, D), lambda i: (0, 0)),
            pl.BlockSpec((1, D), lambda i: (0, 0)),
        ],
        out_specs=pl.BlockSpec((BN, D), lambda i: (i, 0)),
        out_shape=jax.ShapeDtypeStruct((N, D), jnp.float32),
    )(node_features, parts, nm_w1, nm_b1.reshape(1, D), nm_w2, nm_b2.reshape(1, D))

    return node_out, edge_out
